# baseline (speedup 1.0000x reference)
import functools

import jax
import jax.numpy as jnp
from jax import lax
from jax.experimental import pallas as pl
from jax.experimental.pallas import tpu as pltpu
from jax.experimental.pallas import tpu_sc as plsc

BATCH = 4096
SEQ = 200
N_TOKENS = 100
EMBED_DIM = 32
SEQ_OUT = N_TOKENS + SEQ

NUM_CORES = 2
NUM_SUBCORES = 16
NW = NUM_CORES * NUM_SUBCORES
B_PER_W = BATCH // NW
G = 4
ROUNDS = B_PER_W // G
CHUNK = 100
NCHUNK = SEQ // CHUNK
NQ = SEQ * EMBED_DIM // 128

QL = N_TOKENS // 4
QTOT = SEQ_OUT // 4
BBLK = 2048
NB = BATCH // BBLK


def _gather_body(ids_hbm, table_hbm, out_hbm, idx_v, stage_v, chunk_v,
                 sem0, sem1):
    wid = lax.axis_index("s") * NUM_CORES + lax.axis_index("c")
    base = wid * B_PER_W
    sems = (sem0, sem1)

    def fire(buf, r):
        b0 = base + r * G
        pltpu.sync_copy(ids_hbm.at[pl.ds(b0, G)], idx_v.at[buf])
        for i in range(G):
            for j in range(NCHUNK):
                pltpu.async_copy(
                    table_hbm.at[idx_v.at[buf, i, j]],
                    stage_v.at[buf, i, pl.ds(j * CHUNK, CHUNK)],
                    sems[buf])

    def drain(buf):
        for i in range(G):
            for j in range(NCHUNK):
                pltpu.make_async_copy(
                    table_hbm.at[idx_v.at[buf, i, j]],
                    stage_v.at[buf, i, pl.ds(j * CHUNK, CHUNK)],
                    sems[buf]).wait()

    fire(0, 0)

    def outer(rr, carry):
        for b in range(2):
            r = rr * 2 + b

            @pl.when(r + 1 < ROUNDS)
            def _():
                fire(1 - b, r + 1)

            drain(b)

            def rechunk(i, carry2):
                for qg in range(NQ):
                    for sl in range(4):
                        for h in range(2):
                            chunk_v[b, qg, i, pl.ds(sl * 32 + h * 16, 16)] = (
                                stage_v[b, i, qg * 4 + sl, pl.ds(h * 16, 16)])
                return carry2

            lax.fori_loop(0, G, rechunk, 0)
            pltpu.sync_copy(chunk_v.at[b],
                            out_hbm.at[:, pl.ds(base + r * G, G)])
        return carry

    lax.fori_loop(0, ROUNDS // 2, outer, 0)


def _concat_body(learned_ref, g_ref, o_ref):
    q = pl.program_id(0)

    @pl.when(q < QL)
    def _():
        l = learned_ref[pl.ds(4 * q, 4), :]
        o_ref[...] = jnp.broadcast_to(l[:, :, None], (4, EMBED_DIM, BBLK))

    @pl.when(q >= QL)
    def _():
        x = g_ref[...].reshape(BBLK, 128)
        o_ref[...] = x.T.reshape(4, EMBED_DIM, BBLK)


@jax.jit
def _soft_embedding(ids3, wte_weight, learned_embedding):
    mesh = plsc.VectorSubcoreMesh(core_axis_name="c", subcore_axis_name="s",
                                  num_cores=NUM_CORES,
                                  num_subcores=NUM_SUBCORES)
    gather_fn = functools.partial(
        pl.kernel,
        out_type=jax.ShapeDtypeStruct((NQ, BATCH, 128), jnp.float32),
        mesh=mesh,
        scratch_types=[
            pltpu.VMEM((2, G, NCHUNK, CHUNK), jnp.int32),
            pltpu.VMEM((2, G, SEQ, EMBED_DIM), jnp.float32),
            pltpu.VMEM((2, NQ, G, 128), jnp.float32),
            pltpu.SemaphoreType.DMA,
            pltpu.SemaphoreType.DMA,
        ],
        compiler_params=pltpu.CompilerParams(use_tc_tiling_on_sc=False),
    )(_gather_body)
    g5 = gather_fn(ids3, wte_weight)

    o = pl.pallas_call(
        _concat_body,
        grid=(QTOT, NB),
        in_specs=[
            pl.BlockSpec((N_TOKENS, EMBED_DIM), lambda q, b: (0, 0)),
            pl.BlockSpec((1, BBLK, 128),
                         lambda q, b: (jnp.maximum(q - QL, 0), b, 0)),
        ],
        out_specs=pl.BlockSpec((4, EMBED_DIM, BBLK),
                               lambda q, b: (q, 0, b)),
        out_shape=jax.ShapeDtypeStruct((SEQ_OUT, EMBED_DIM, BATCH),
                                       jnp.float32),
    )(learned_embedding, g5)
    return jnp.transpose(o, (2, 0, 1))


def kernel(input_ids, wte_weight, learned_embedding):
    ids3 = input_ids.astype(jnp.int32).reshape(BATCH, NCHUNK, CHUNK)
    return _soft_embedding(ids3, wte_weight, learned_embedding)


# speedup vs baseline: 1.0364x; 1.0364x over previous
import functools

import jax
import jax.numpy as jnp
from jax import lax
from jax.experimental import pallas as pl
from jax.experimental.pallas import tpu as pltpu
from jax.experimental.pallas import tpu_sc as plsc

BATCH = 4096
SEQ = 200
N_TOKENS = 100
EMBED_DIM = 32
SEQ_OUT = N_TOKENS + SEQ

NUM_CORES = 2
NUM_SUBCORES = 16
NW = NUM_CORES * NUM_SUBCORES
B_PER_W = BATCH // NW
G = 4
ROUNDS = B_PER_W // G
CHUNK = 100
NCHUNK = SEQ // CHUNK
NQ = SEQ * EMBED_DIM // 128

QL = N_TOKENS // 4
QTOT = SEQ_OUT // 4
BBLK = 2048
NB = BATCH // BBLK


def _gather_body(ids_hbm, table_hbm, out_hbm, idx_v, stage_v, chunk_v,
                 sem0, sem1, wsem0, wsem1):
    wid = lax.axis_index("s") * NUM_CORES + lax.axis_index("c")
    base = wid * B_PER_W
    sems = (sem0, sem1)
    wsems = (wsem0, wsem1)

    def fire(buf, r):
        b0 = base + r * G
        pltpu.sync_copy(ids_hbm.at[pl.ds(b0, G)], idx_v.at[buf])
        for i in range(G):
            for j in range(NCHUNK):
                pltpu.async_copy(
                    table_hbm.at[idx_v.at[buf, i, j]],
                    stage_v.at[buf, i, pl.ds(j * CHUNK, CHUNK)],
                    sems[buf])

    def drain(buf):
        for i in range(G):
            for j in range(NCHUNK):
                pltpu.make_async_copy(
                    table_hbm.at[idx_v.at[buf, i, j]],
                    stage_v.at[buf, i, pl.ds(j * CHUNK, CHUNK)],
                    sems[buf]).wait()

    fire(0, 0)

    def outer(rr, carry):
        for b in range(2):
            r = rr * 2 + b

            @pl.when(r + 1 < ROUNDS)
            def _():
                fire(1 - b, r + 1)

            drain(b)

            def rechunk(i, carry2):
                for qg in range(NQ):
                    for sl in range(4):
                        for h in range(2):
                            chunk_v[b, qg, i, pl.ds(sl * 32 + h * 16, 16)] = (
                                stage_v[b, i, qg * 4 + sl, pl.ds(h * 16, 16)])
                return carry2

            @pl.when(r >= 2)
            def _():
                pltpu.make_async_copy(
                    chunk_v.at[b],
                    out_hbm.at[:, pl.ds(base + (r - 2) * G, G)],
                    wsems[b]).wait()

            lax.fori_loop(0, G, rechunk, 0)
            pltpu.async_copy(chunk_v.at[b],
                             out_hbm.at[:, pl.ds(base + r * G, G)],
                             wsems[b])
        return carry

    lax.fori_loop(0, ROUNDS // 2, outer, 0)
    for b in range(2):
        r = ROUNDS - 2 + b
        pltpu.make_async_copy(
            chunk_v.at[b],
            out_hbm.at[:, pl.ds(base + r * G, G)],
            wsems[b]).wait()


def _concat_body(learned_ref, g_ref, o_ref):
    q = pl.program_id(0)

    @pl.when(q < QL)
    def _():
        l = learned_ref[pl.ds(4 * q, 4), :]
        o_ref[...] = jnp.broadcast_to(l[:, :, None], (4, EMBED_DIM, BBLK))

    @pl.when(q >= QL)
    def _():
        x = g_ref[...].reshape(BBLK, 128)
        o_ref[...] = x.T.reshape(4, EMBED_DIM, BBLK)


@jax.jit
def _soft_embedding(ids3, wte_weight, learned_embedding):
    mesh = plsc.VectorSubcoreMesh(core_axis_name="c", subcore_axis_name="s",
                                  num_cores=NUM_CORES,
                                  num_subcores=NUM_SUBCORES)
    gather_fn = functools.partial(
        pl.kernel,
        out_type=jax.ShapeDtypeStruct((NQ, BATCH, 128), jnp.float32),
        mesh=mesh,
        scratch_types=[
            pltpu.VMEM((2, G, NCHUNK, CHUNK), jnp.int32),
            pltpu.VMEM((2, G, SEQ, EMBED_DIM), jnp.float32),
            pltpu.VMEM((2, NQ, G, 128), jnp.float32),
            pltpu.SemaphoreType.DMA,
            pltpu.SemaphoreType.DMA,
            pltpu.SemaphoreType.DMA,
            pltpu.SemaphoreType.DMA,
        ],
        compiler_params=pltpu.CompilerParams(use_tc_tiling_on_sc=False),
    )(_gather_body)
    g5 = gather_fn(ids3, wte_weight)

    o = pl.pallas_call(
        _concat_body,
        grid=(QTOT, NB),
        in_specs=[
            pl.BlockSpec((N_TOKENS, EMBED_DIM), lambda q, b: (0, 0)),
            pl.BlockSpec((1, BBLK, 128),
                         lambda q, b: (jnp.maximum(q - QL, 0), b, 0)),
        ],
        out_specs=pl.BlockSpec((4, EMBED_DIM, BBLK),
                               lambda q, b: (q, 0, b)),
        out_shape=jax.ShapeDtypeStruct((SEQ_OUT, EMBED_DIM, BATCH),
                                       jnp.float32),
    )(learned_embedding, g5)
    return jnp.transpose(o, (2, 0, 1))


def kernel(input_ids, wte_weight, learned_embedding):
    ids3 = input_ids.astype(jnp.int32).reshape(BATCH, NCHUNK, CHUNK)
    return _soft_embedding(ids3, wte_weight, learned_embedding)


# speedup vs baseline: 1.1169x; 1.0777x over previous
import functools

import jax
import jax.numpy as jnp
from jax import lax
from jax.experimental import pallas as pl
from jax.experimental.pallas import tpu as pltpu
from jax.experimental.pallas import tpu_sc as plsc

BATCH = 4096
SEQ = 200
N_TOKENS = 100
EMBED_DIM = 32
SEQ_OUT = N_TOKENS + SEQ

NUM_CORES = 2
NUM_SUBCORES = 16
NW = NUM_CORES * NUM_SUBCORES
B_PER_W = BATCH // NW
G = 4
ROUNDS = B_PER_W // G
CHUNK = 100
NCHUNK = SEQ // CHUNK
NQ = SEQ * EMBED_DIM // 128

QL = N_TOKENS // 4
QTOT = SEQ_OUT // 4
BBLK = 4096
NB = BATCH // BBLK


def _gather_body(ids_hbm, table_hbm, out_hbm, idx_v, stage_v, chunk_v,
                 sem0, sem1, wsem0, wsem1):
    wid = lax.axis_index("s") * NUM_CORES + lax.axis_index("c")
    base = wid * B_PER_W
    sems = (sem0, sem1)
    wsems = (wsem0, wsem1)

    def fire(buf, r):
        b0 = base + r * G
        pltpu.sync_copy(ids_hbm.at[pl.ds(b0, G)], idx_v.at[buf])
        for i in range(G):
            for j in range(NCHUNK):
                pltpu.async_copy(
                    table_hbm.at[idx_v.at[buf, i, j]],
                    stage_v.at[buf, i, pl.ds(j * CHUNK, CHUNK)],
                    sems[buf])

    def drain(buf):
        for i in range(G):
            for j in range(NCHUNK):
                pltpu.make_async_copy(
                    table_hbm.at[idx_v.at[buf, i, j]],
                    stage_v.at[buf, i, pl.ds(j * CHUNK, CHUNK)],
                    sems[buf]).wait()

    fire(0, 0)

    def outer(rr, carry):
        for b in range(2):
            r = rr * 2 + b

            @pl.when(r + 1 < ROUNDS)
            def _():
                fire(1 - b, r + 1)

            drain(b)

            def rechunk(i, carry2):
                for qg in range(NQ):
                    for sl in range(4):
                        for h in range(2):
                            chunk_v[b, qg, i, pl.ds(sl * 32 + h * 16, 16)] = (
                                stage_v[b, i, qg * 4 + sl, pl.ds(h * 16, 16)])
                return carry2

            @pl.when(r >= 2)
            def _():
                pltpu.make_async_copy(
                    chunk_v.at[b],
                    out_hbm.at[:, pl.ds(base + (r - 2) * G, G)],
                    wsems[b]).wait()

            lax.fori_loop(0, G, rechunk, 0)
            pltpu.async_copy(chunk_v.at[b],
                             out_hbm.at[:, pl.ds(base + r * G, G)],
                             wsems[b])
        return carry

    lax.fori_loop(0, ROUNDS // 2, outer, 0)
    for b in range(2):
        r = ROUNDS - 2 + b
        pltpu.make_async_copy(
            chunk_v.at[b],
            out_hbm.at[:, pl.ds(base + r * G, G)],
            wsems[b]).wait()


def _concat_body(learned_ref, g_ref, o_ref):
    q = pl.program_id(0)

    @pl.when(q < QL)
    def _():
        l = learned_ref[pl.ds(4 * q, 4), :]
        o_ref[...] = jnp.broadcast_to(l[:, :, None], (4, EMBED_DIM, BBLK))

    @pl.when(q >= QL)
    def _():
        x = g_ref[...].reshape(BBLK, 128)
        o_ref[...] = x.T.reshape(4, EMBED_DIM, BBLK)


@jax.jit
def _soft_embedding(ids3, wte_weight, learned_embedding):
    mesh = plsc.VectorSubcoreMesh(core_axis_name="c", subcore_axis_name="s",
                                  num_cores=NUM_CORES,
                                  num_subcores=NUM_SUBCORES)
    gather_fn = functools.partial(
        pl.kernel,
        out_type=jax.ShapeDtypeStruct((NQ, BATCH, 128), jnp.float32),
        mesh=mesh,
        scratch_types=[
            pltpu.VMEM((2, G, NCHUNK, CHUNK), jnp.int32),
            pltpu.VMEM((2, G, SEQ, EMBED_DIM), jnp.float32),
            pltpu.VMEM((2, NQ, G, 128), jnp.float32),
            pltpu.SemaphoreType.DMA,
            pltpu.SemaphoreType.DMA,
            pltpu.SemaphoreType.DMA,
            pltpu.SemaphoreType.DMA,
        ],
        compiler_params=pltpu.CompilerParams(use_tc_tiling_on_sc=False),
    )(_gather_body)
    g5 = gather_fn(ids3, wte_weight)

    o = pl.pallas_call(
        _concat_body,
        grid=(QTOT, NB),
        in_specs=[
            pl.BlockSpec((N_TOKENS, EMBED_DIM), lambda q, b: (0, 0)),
            pl.BlockSpec((1, BBLK, 128),
                         lambda q, b: (jnp.maximum(q - QL, 0), b, 0)),
        ],
        out_specs=pl.BlockSpec((4, EMBED_DIM, BBLK),
                               lambda q, b: (q, 0, b)),
        out_shape=jax.ShapeDtypeStruct((SEQ_OUT, EMBED_DIM, BATCH),
                                       jnp.float32),
    )(learned_embedding, g5)
    return jnp.transpose(o, (2, 0, 1))


def kernel(input_ids, wte_weight, learned_embedding):
    ids3 = input_ids.astype(jnp.int32).reshape(BATCH, NCHUNK, CHUNK)
    return _soft_embedding(ids3, wte_weight, learned_embedding)
